# baseline (speedup 1.0000x reference)
import functools

import jax
import jax.numpy as jnp
from jax import lax
from jax.experimental import pallas as pl
from jax.experimental.pallas import tpu as pltpu
from jax.experimental.pallas import tpu_sc as plsc

_CHUNK = 128


def _make_gather(V, D, B):
    info = plsc.get_sparse_core_info()
    NC, NS = info.num_cores, info.num_subcores
    NW = NC * NS
    assert B % (NW * _CHUNK) == 0
    b_per_w = B // NW
    n_chunks = b_per_w // _CHUNK

    mesh = plsc.VectorSubcoreMesh(core_axis_name="c", subcore_axis_name="s")

    @functools.partial(
        pl.kernel,
        mesh=mesh,
        out_type=jax.ShapeDtypeStruct((B, D), jnp.float32),
        scratch_types=[
            pltpu.VMEM((n_chunks, _CHUNK), jnp.int32),
            pltpu.VMEM((b_per_w, D), jnp.float32),
            pltpu.SemaphoreType.DMA,
        ],
        compiler_params=pltpu.CompilerParams(use_tc_tiling_on_sc=False),
    )
    def gather_kernel(labels_hbm, table_hbm, out_hbm, idx_v, rows_v, sem):
        wid = lax.axis_index("s") * NC + lax.axis_index("c")
        base = wid * b_per_w
        pltpu.sync_copy(labels_hbm.at[wid], idx_v)
        copies = []
        for j in range(n_chunks):
            copies.append(
                pltpu.async_copy(
                    table_hbm.at[idx_v.at[j]],
                    rows_v.at[pl.ds(j * _CHUNK, _CHUNK)],
                    sem,
                )
            )
        for c in copies:
            c.wait()
        pltpu.sync_copy(rows_v, out_hbm.at[pl.ds(base, b_per_w)])

    return gather_kernel


@jax.jit
def kernel(labels, embedding_weight):
    V, D = embedding_weight.shape
    (B,) = labels.shape
    info = plsc.get_sparse_core_info()
    NW = info.num_cores * info.num_subcores
    labels3d = labels.astype(jnp.int32).reshape(NW, B // NW // _CHUNK, _CHUNK)
    return _make_gather(V, D, B)(labels3d, embedding_weight)


# speedup vs baseline: 1.5803x; 1.5803x over previous
import functools

import jax
import jax.numpy as jnp
from jax import lax
from jax.experimental import pallas as pl
from jax.experimental.pallas import tpu as pltpu
from jax.experimental.pallas import tpu_sc as plsc


def _make_gather(V, D, B):
    info = plsc.get_sparse_core_info()
    NC, NS, L = info.num_cores, info.num_subcores, info.num_lanes
    NW = NC * NS
    assert B % (NW * L) == 0 and D % L == 0
    b_per_w = B // NW
    n_chunks = b_per_w // L

    mesh = plsc.VectorSubcoreMesh(core_axis_name="c", subcore_axis_name="s")

    @functools.partial(
        pl.kernel,
        mesh=mesh,
        out_type=jax.ShapeDtypeStruct((D, B), jnp.float32),
        scratch_types=[
            pltpu.VMEM((b_per_w,), jnp.int32),
            pltpu.VMEM((2, L, 8, D), jnp.float32),
            pltpu.VMEM((D, b_per_w), jnp.float32),
            pltpu.SemaphoreType.DMA,
            pltpu.SemaphoreType.DMA,
        ],
        compiler_params=pltpu.CompilerParams(needs_layout_passes=False),
    )
    def gather_kernel(
        labels_hbm, table_hbm, out_t_hbm, lab_v, slots_v, colt_v, sem0, sem1
    ):
        wid = lax.axis_index("s") * NC + lax.axis_index("c")
        base = wid * b_per_w
        pltpu.sync_copy(labels_hbm.at[pl.ds(base, b_per_w)], lab_v)

        lane_iota = lax.iota(jnp.int32, L)
        drain_src = table_hbm.at[pl.ds(0, 8 * L), :].reshape(L, 8, D)

        def fire(c, p):
            labs = lab_v[pl.ds(c * L, L)]
            sem = sem0 if p == 0 else sem1
            for lane in range(L):
                blk = pl.multiple_of((labs[lane] >> 3) << 3, 8)
                pltpu.async_copy(
                    table_hbm.at[pl.ds(blk, 8), :], slots_v.at[p, lane], sem
                )

        def drain_extract(c, p):
            pltpu.make_async_copy(
                drain_src, slots_v.at[p], sem0 if p == 0 else sem1
            ).wait()
            labs = lab_v[pl.ds(c * L, L)]
            for lane in range(L):
                r = labs[lane] & 7
                colv = jnp.full((L,), c * L + lane, dtype=jnp.int32)
                for k in range(D // L):
                    v = slots_v[p, lane, r, pl.ds(k * L, L)]
                    plsc.store_scatter(colt_v, [lane_iota + k * L, colv], v)

        fire(0, 0)
        fire(1, 1)

        def loop_body(c, _):
            drain_extract(2 * c, 0)
            fire(2 * c + 2, 0)
            drain_extract(2 * c + 1, 1)
            fire(2 * c + 3, 1)
            return ()

        lax.fori_loop(0, n_chunks // 2 - 1, loop_body, ())
        drain_extract(n_chunks - 2, 0)
        drain_extract(n_chunks - 1, 1)
        pltpu.sync_copy(colt_v, out_t_hbm.at[:, pl.ds(base, b_per_w)])

    return gather_kernel


@jax.jit
def kernel(labels, embedding_weight):
    V, D = embedding_weight.shape
    (B,) = labels.shape
    out_t = _make_gather(V, D, B)(labels.astype(jnp.int32), embedding_weight)
    return out_t.T


# speedup vs baseline: 2.5116x; 1.5894x over previous
import functools

import jax
import jax.numpy as jnp
from jax import lax
from jax.experimental import pallas as pl
from jax.experimental.pallas import tpu as pltpu
from jax.experimental.pallas import tpu_sc as plsc

_CL = 4


def _make_gather(V, D, B):
    info = plsc.get_sparse_core_info()
    NC, NS, L = info.num_cores, info.num_subcores, info.num_lanes
    NW = NC * NS
    assert B % (NW * L) == 0 and D % L == 0
    b_per_w = B // NW
    n_groups = b_per_w // L

    mesh = plsc.VectorSubcoreMesh(core_axis_name="c", subcore_axis_name="s")

    @functools.partial(
        pl.kernel,
        mesh=mesh,
        out_type=jax.ShapeDtypeStruct((D, B), jnp.float32),
        scratch_types=[
            pltpu.VMEM((b_per_w,), jnp.int32),
            pltpu.VMEM((2, _CL, D, 128), jnp.float32),
            pltpu.VMEM((D, b_per_w), jnp.float32),
            pltpu.SemaphoreType.DMA,
            pltpu.SemaphoreType.DMA,
        ],
        compiler_params=pltpu.CompilerParams(needs_layout_passes=False),
    )
    def gather_kernel(
        labels_hbm, table_t_hbm, out_t_hbm, lab_v, slots_v, colt_v, sem0, sem1
    ):
        wid = lax.axis_index("s") * NC + lax.axis_index("c")
        base = wid * b_per_w
        pltpu.sync_copy(labels_hbm.at[pl.ds(base, b_per_w)], lab_v)

        lane_iota = lax.iota(jnp.int32, L)
        drain_src = table_t_hbm.at[:, pl.ds(0, 128)]

        def fire(labs, s, p):
            sem = sem0 if p == 0 else sem1
            for j in range(_CL):
                tcol = pl.multiple_of((labs[s * _CL + j] >> 7) << 7, 128)
                pltpu.async_copy(
                    table_t_hbm.at[:, pl.ds(tcol, 128)], slots_v.at[p, j], sem
                )

        def extract(labs, s, col0, p):
            sem = sem0 if p == 0 else sem1
            for j in range(_CL):
                pltpu.make_async_copy(drain_src, slots_v.at[p, j], sem).wait()
            for j in range(_CL):
                c = labs[s * _CL + j] & 127
                cvec = jnp.full((L,), c, dtype=jnp.int32)
                ovec = jnp.full((L,), col0 + j, dtype=jnp.int32)
                for q in range(D // L):
                    rows = lane_iota + q * L
                    v = plsc.load_gather(slots_v.at[p, j], [rows, cvec])
                    plsc.store_scatter(colt_v, [rows, ovec], v)

        cpg = L // _CL
        labs0 = lab_v[pl.ds(0, L)]
        fire(labs0, 0, 0)
        fire(labs0, 1, 1)
        extract(labs0, 0, 0, 0)
        fire(labs0, 2, 0)
        extract(labs0, 1, _CL, 1)
        fire(labs0, 3, 1)

        def loop_body(g, labs_prev):
            labs = lab_v[pl.ds(g * L, L)]
            k0 = g * cpg
            extract(labs_prev, 2, (k0 - 2) * _CL, 0)
            fire(labs, 0, 0)
            extract(labs_prev, 3, (k0 - 1) * _CL, 1)
            fire(labs, 1, 1)
            extract(labs, 0, k0 * _CL, 0)
            fire(labs, 2, 0)
            extract(labs, 1, (k0 + 1) * _CL, 1)
            fire(labs, 3, 1)
            return labs

        labs_last = lax.fori_loop(1, n_groups, loop_body, labs0)
        n_chunks = n_groups * cpg
        extract(labs_last, 2, (n_chunks - 2) * _CL, 0)
        extract(labs_last, 3, (n_chunks - 1) * _CL, 1)

        pltpu.sync_copy(colt_v, out_t_hbm.at[:, pl.ds(base, b_per_w)])

    return gather_kernel


@jax.jit
def kernel(labels, embedding_weight):
    V, D = embedding_weight.shape
    (B,) = labels.shape
    out_t = _make_gather(V, D, B)(labels.astype(jnp.int32), embedding_weight.T)
    return out_t.T
